# baseline (speedup 1.0000x reference)
import functools

import jax
import jax.numpy as jnp
from jax.experimental import pallas as pl

_TILE = 1024
_K = 8

_DN_T = (((1,), (1,)), ((), ()))


def _body(pr_ref, pa_ref, fr_ref, fa_ref, W1_ref, b1_ref, W2_ref, b2_ref,
          bp_ref, enh_ref):
    p_row = pr_ref[0]
    p_all = pa_ref[0]
    f_r = fr_ref[0]
    f_a = fa_ref[0]

    N = pa_ref.shape[1]
    D = fa_ref.shape[2]
    one3 = jnp.ones((1, 3), jnp.float32)
    oneD = jnp.ones((1, D), jnp.float32)

    pp = jax.lax.dot_general(p_row, p_all, _DN_T,
                             preferred_element_type=jnp.float32)
    pn_row = jnp.sum(p_row * p_row, axis=1, keepdims=True)
    pn_all = jax.lax.dot_general(one3, p_all * p_all, _DN_T,
                                 preferred_element_type=jnp.float32)
    d2 = (pn_row + pn_all) - 2.0 * pp

    big = jnp.float32(3e38)
    work = d2
    m = None
    for k in range(_K):
        m = jnp.min(work, axis=1, keepdims=True)
        if k < _K - 1:
            work = jnp.where(work <= m, big, work)
    mask = d2 <= m

    fn_all = jax.lax.dot_general(oneD, f_a * f_a, _DN_T,
                                 preferred_element_type=jnp.float32)
    fn_row = jnp.sum(f_r * f_r, axis=1, keepdims=True)
    gram = jax.lax.dot_general(f_r, f_a, _DN_T,
                               preferred_element_type=jnp.float32)
    fd2 = jnp.maximum(fn_row + (fn_all - 2.0 * gram), 0.0)
    fd = jnp.sqrt(fd2)
    oneN = jnp.ones((N, 1), jnp.float32)
    acc = jax.lax.dot(jnp.where(mask, fd, 0.0), oneN,
                      preferred_element_type=jnp.float32)
    fg = acc * (1.0 / _K)

    h = jnp.maximum(
        jax.lax.dot(f_r, W1_ref[...], preferred_element_type=jnp.float32,
                    precision=jax.lax.Precision.HIGHEST) + b1_ref[...], 0.0)
    logits = jax.lax.dot(h, W2_ref[...], preferred_element_type=jnp.float32,
                         precision=jax.lax.Precision.HIGHEST) + b2_ref[...]
    bp = jax.nn.sigmoid(logits)

    enh = f_r + 0.3 * (jnp.tanh(5.0 * fg) * bp)
    bp_ref[0] = bp
    enh_ref[0] = enh


@jax.jit
def kernel(features, points, W1, b1, W2, b2):
    B, N, D = features.shape
    b1r = b1.reshape(1, -1)
    W2r = W2.reshape(-1, 1)
    b2r = b2.reshape(1, 1)

    grid = (B, N // _TILE)
    bp, enh = pl.pallas_call(
        _body,
        grid=grid,
        in_specs=[
            pl.BlockSpec((1, _TILE, 3), lambda b, t: (b, t, 0)),
            pl.BlockSpec((1, N, 3), lambda b, t: (b, 0, 0)),
            pl.BlockSpec((1, _TILE, D), lambda b, t: (b, t, 0)),
            pl.BlockSpec((1, N, D), lambda b, t: (b, 0, 0)),
            pl.BlockSpec((D, 64), lambda b, t: (0, 0)),
            pl.BlockSpec((1, 64), lambda b, t: (0, 0)),
            pl.BlockSpec((64, 1), lambda b, t: (0, 0)),
            pl.BlockSpec((1, 1), lambda b, t: (0, 0)),
        ],
        out_specs=[
            pl.BlockSpec((1, _TILE, 1), lambda b, t: (b, t, 0)),
            pl.BlockSpec((1, _TILE, D), lambda b, t: (b, t, 0)),
        ],
        out_shape=[
            jax.ShapeDtypeStruct((B, N, 1), jnp.float32),
            jax.ShapeDtypeStruct((B, N, D), jnp.float32),
        ],
    )(points, points, features, features, W1, b1r, W2r, b2r)
    return (bp, enh)
